# baseline (speedup 1.0000x reference)
import math

import jax
import jax.numpy as jnp
from jax.experimental import pallas as pl
from jax.experimental.pallas import tpu as pltpu

NU = 1024
NI = 4096
NP = 64
K = 20
DELTA = 1e-8
UBLK = 256
NBLK = NU // UBLK

_DISCOUNTS = [1.0 / math.log2(r + 2) for r in range(K)]


def _mm(a, b):
    return jax.lax.dot_general(a, b, (((1,), (0,)), ((), ())),
                               preferred_element_type=jnp.float32)


def _body(scores_ref, rating_ref, pid_ref, eW1_ref, eb1_ref, eW2_ref,
          eb2_ref, eW3_ref, eb3_ref, pW1_ref, pb1_ref, pW2_ref, pb2_ref,
          uW1_ref, ub1_ref, uW2_ref, ub2_ref,
          expo_ref, ipw_ref, uf_ref, loss_ref,
          rsum_ref, ufs_ref):
    i = pl.program_id(0)

    @pl.when(i < NBLK)
    def _topk_phase():
        scores = scores_ref[...]
        rating = rating_ref[...]
        seen = rating > 0.0
        masked = jnp.where(seen, jnp.float32(-1e10), scores)
        hist = jnp.sum(seen.astype(jnp.float32), axis=1, keepdims=True)
        recip = 1.0 / jnp.maximum(hist, 1.0)
        ufs_ref[pl.ds(i * UBLK, UBLK), :] = recip

        @pl.when(i == 0)
        def _():
            rsum_ref[0] = 0.0

        rsum_ref[0] += jnp.sum(recip)

        iota = jax.lax.broadcasted_iota(jnp.int32, (UBLK, NI), 1)
        dacc = jnp.zeros((UBLK, NI), jnp.float32)
        for r in range(K):
            m = jnp.max(masked, axis=1, keepdims=True)
            eq = masked == m
            idx = jnp.min(jnp.where(eq, iota, NI), axis=1, keepdims=True)
            oh = iota == idx
            dacc = jnp.where(oh, jnp.float32(_DISCOUNTS[r]), dacc)
            masked = jnp.where(oh, -jnp.inf, masked)
        expo = jnp.sum(dacc, axis=0, keepdims=True)

        @pl.when(i == 0)
        def _():
            expo_ref[...] = expo

        @pl.when(i != 0)
        def _():
            expo_ref[...] += expo

        @pl.when(i == NBLK - 1)
        def _():
            pid = pid_ref[...]
            piota = jax.lax.broadcasted_iota(jnp.int32, (NP, NI), 0)
            onehot = pid == piota
            full_expo = expo_ref[...]
            pe = jnp.sum(jnp.where(onehot, full_expo, 0.0), axis=1,
                         keepdims=True)
            cnt = jnp.sum(onehot.astype(jnp.float32), axis=1, keepdims=True)
            pavg = pe / jnp.maximum(cnt, 1.0)
            pf = 1.0 / jnp.maximum(pavg + DELTA, DELTA)
            pf = pf / jnp.mean(pf)
            ipw = jnp.sum(jnp.where(onehot, pf, 0.0), axis=0, keepdims=True)
            ipw_ref[...] = ipw / jnp.mean(ipw)
            ufs_ref[...] = ufs_ref[...] * (NU / rsum_ref[0])

    @pl.when(i >= NBLK)
    def _mlp_phase():
        j = i - NBLK
        relu = jax.nn.relu
        x = rating_ref[...]
        uf = ufs_ref[pl.ds(j * UBLK, UBLK), :]
        uf_ref[...] = uf
        h1 = relu(_mm(x, eW1_ref[...]) + eb1_ref[...])
        h2 = relu(_mm(h1, eW2_ref[...]) + eb2_ref[...])
        h_enc = (_mm(h2, eW3_ref[...]) + eb3_ref[...]) * x

        pW1, pb1 = pW1_ref[...], pb1_ref[...]
        pW2, pb2 = pW2_ref[...], pb2_ref[...]
        uW1, ub1 = uW1_ref[...], ub1_ref[...]
        uW2, ub2 = uW2_ref[...], ub2_ref[...]

        h_p = _mm(relu(_mm(h_enc, pW1) + pb1), pW2) + pb2
        t_p = _mm(relu(_mm(ipw_ref[...] * x, pW1) + pb1), pW2) + pb2
        h_u = _mm(relu(_mm(h_enc, uW1) + ub1), uW2) + ub2
        t_u = _mm(relu(_mm(uf * x, uW1) + ub1), uW2) + ub2

        blk = (jnp.sum((h_p - t_p) ** 2)
               + jnp.sum((h_u - t_u) ** 2)) / (NU * NI)
        blk = jnp.full((1, 1), 1.0, jnp.float32) * blk

        @pl.when(j == 0)
        def _():
            loss_ref[...] = blk

        @pl.when(j != 0)
        def _():
            loss_ref[...] += blk


def kernel(scores, rating_matrix, eW1, eb1, eW2, eb2, eW3, eb3, pW1, pb1,
           pW2, pb2, uW1, ub1, uW2, ub2, provider_ids):
    pid2 = provider_ids.reshape(1, NI).astype(jnp.int32)
    full = lambda shape: pl.BlockSpec(shape, lambda i: (0,) * len(shape))
    expo, ipw, uf, loss = pl.pallas_call(
        _body,
        grid=(2 * NBLK,),
        in_specs=[
            pl.BlockSpec((UBLK, NI), lambda i: (jnp.minimum(i, NBLK - 1), 0)),
            pl.BlockSpec((UBLK, NI), lambda i: (jax.lax.rem(i, NBLK), 0)),
            full((1, NI)),
            full((NI, 256)), full((1, 256)),
            full((256, 128)), full((1, 128)),
            full((128, NI)), full((1, NI)),
            full((NI, 128)), full((1, 128)),
            full((128, NI)), full((1, NI)),
            full((NI, 128)), full((1, 128)),
            full((128, NI)), full((1, NI)),
        ],
        out_specs=[
            pl.BlockSpec((1, NI), lambda i: (0, 0)),
            pl.BlockSpec((1, NI), lambda i: (0, 0)),
            pl.BlockSpec((UBLK, 1), lambda i: (jnp.maximum(i - NBLK, 0), 0)),
            pl.BlockSpec((1, 1), lambda i: (0, 0)),
        ],
        out_shape=[
            jax.ShapeDtypeStruct((1, NI), jnp.float32),
            jax.ShapeDtypeStruct((1, NI), jnp.float32),
            jax.ShapeDtypeStruct((NU, 1), jnp.float32),
            jax.ShapeDtypeStruct((1, 1), jnp.float32),
        ],
        scratch_shapes=[
            pltpu.SMEM((1,), jnp.float32),
            pltpu.VMEM((NU, 1), jnp.float32),
        ],
    )(scores, rating_matrix, pid2, eW1, eb1.reshape(1, -1), eW2,
      eb2.reshape(1, -1), eW3, eb3.reshape(1, -1), pW1, pb1.reshape(1, -1),
      pW2, pb2.reshape(1, -1), uW1, ub1.reshape(1, -1), uW2,
      ub2.reshape(1, -1))

    return (loss.reshape(()), ipw.reshape(NI), uf.reshape(NU),
            expo.reshape(NI))


# speedup vs baseline: 1.7340x; 1.7340x over previous
import math

import jax
import jax.numpy as jnp
from jax.experimental import pallas as pl
from jax.experimental.pallas import tpu as pltpu

NU = 1024
NI = 4096
NP = 64
K = 20
DELTA = 1e-8
UBLK = 256

_DISCOUNTS = [1.0 / math.log2(r + 2) for r in range(K)]


def _topk_fair_body(scores_ref, rating_ref, pid_ref, expo_ref, recip_ref,
                    rmean_ref, ipw_ref, rsum_ref):
    i = pl.program_id(0)
    scores = scores_ref[...]
    rating = rating_ref[...]
    seen = rating > 0.0
    masked = jnp.where(seen, jnp.float32(-1e10), scores)
    hist = jnp.sum(seen.astype(jnp.float32), axis=1, keepdims=True)
    recip = 1.0 / jnp.maximum(hist, 1.0)
    recip_ref[...] = recip

    @pl.when(i == 0)
    def _():
        rsum_ref[0] = 0.0

    rsum_ref[0] += jnp.sum(recip)

    iota_f = jax.lax.broadcasted_iota(jnp.int32, (UBLK, NI), 1).astype(
        jnp.float32)
    for r in range(K):
        m = jnp.max(masked, axis=1, keepdims=True)
        cand = jnp.where(masked == m, iota_f, jnp.float32(NI))
        idxf = jnp.min(cand, axis=1, keepdims=True)
        sel = cand == idxf
        masked = jnp.where(sel, jnp.float32(-(_DISCOUNTS[r] * 2.0**100)),
                           masked)
    dacc = jnp.where(masked <= -1e20, masked * jnp.float32(-(2.0**-100)),
                     0.0)
    expo = jnp.sum(dacc, axis=0, keepdims=True)

    @pl.when(i == 0)
    def _():
        expo_ref[...] = expo

    @pl.when(i != 0)
    def _():
        expo_ref[...] += expo

    @pl.when(i == pl.num_programs(0) - 1)
    def _():
        pid = pid_ref[...]
        piota = jax.lax.broadcasted_iota(jnp.int32, (NP, NI), 0)
        onehot = pid == piota
        full_expo = expo_ref[...]
        pe = jnp.sum(jnp.where(onehot, full_expo, 0.0), axis=1, keepdims=True)
        cnt = jnp.sum(onehot.astype(jnp.float32), axis=1, keepdims=True)
        pavg = pe / jnp.maximum(cnt, 1.0)
        pf = 1.0 / jnp.maximum(pavg + DELTA, DELTA)
        pf = pf / jnp.mean(pf)
        ipw = jnp.sum(jnp.where(onehot, pf, 0.0), axis=0, keepdims=True)
        ipw_ref[...] = ipw / jnp.mean(ipw)
        rmean_ref[...] = jnp.full((1, 1), 1.0 / NU, jnp.float32) * rsum_ref[0]


def _mm(a, b):
    return jax.lax.dot_general(a, b, (((1,), (0,)), ((), ())),
                               preferred_element_type=jnp.float32)


def _mlp_loss_body(x_ref, eW1_ref, eb1_ref, eW2_ref, eb2_ref, eW3_ref,
                   eb3_ref, pW1_ref, pb1_ref, pW2_ref, pb2_ref, uW1_ref,
                   ub1_ref, uW2_ref, ub2_ref, ipw_ref, recip_ref, rmean_ref,
                   loss_ref, uf_ref):
    i = pl.program_id(0)
    relu = jax.nn.relu
    x = x_ref[...]
    uf = recip_ref[...] / rmean_ref[0, 0]
    uf_ref[...] = uf
    h1 = relu(_mm(x, eW1_ref[...]) + eb1_ref[...])
    h2 = relu(_mm(h1, eW2_ref[...]) + eb2_ref[...])
    h_enc = (_mm(h2, eW3_ref[...]) + eb3_ref[...]) * x

    pW1, pb1, pW2, pb2 = pW1_ref[...], pb1_ref[...], pW2_ref[...], pb2_ref[...]
    uW1, ub1, uW2, ub2 = uW1_ref[...], ub1_ref[...], uW2_ref[...], ub2_ref[...]

    h_p = _mm(relu(_mm(h_enc, pW1) + pb1), pW2) + pb2
    t_p = _mm(relu(_mm(ipw_ref[...] * x, pW1) + pb1), pW2) + pb2
    h_u = _mm(relu(_mm(h_enc, uW1) + ub1), uW2) + ub2
    t_u = _mm(relu(_mm(uf * x, uW1) + ub1), uW2) + ub2

    blk = (jnp.sum((h_p - t_p) ** 2) + jnp.sum((h_u - t_u) ** 2)) / (NU * NI)
    blk = jnp.full((1, 1), 1.0, jnp.float32) * blk

    @pl.when(i == 0)
    def _():
        loss_ref[...] = blk

    @pl.when(i != 0)
    def _():
        loss_ref[...] += blk


def kernel(scores, rating_matrix, eW1, eb1, eW2, eb2, eW3, eb3, pW1, pb1,
           pW2, pb2, uW1, ub1, uW2, ub2, provider_ids):
    nblk = NU // UBLK
    pid2 = provider_ids.reshape(1, NI).astype(jnp.int32)
    expo, recip, rmean, ipw = pl.pallas_call(
        _topk_fair_body,
        grid=(nblk,),
        in_specs=[
            pl.BlockSpec((UBLK, NI), lambda i: (i, 0)),
            pl.BlockSpec((UBLK, NI), lambda i: (i, 0)),
            pl.BlockSpec((1, NI), lambda i: (0, 0)),
        ],
        out_specs=[
            pl.BlockSpec((1, NI), lambda i: (0, 0)),
            pl.BlockSpec((UBLK, 1), lambda i: (i, 0)),
            pl.BlockSpec((1, 1), lambda i: (0, 0)),
            pl.BlockSpec((1, NI), lambda i: (0, 0)),
        ],
        out_shape=[
            jax.ShapeDtypeStruct((1, NI), jnp.float32),
            jax.ShapeDtypeStruct((NU, 1), jnp.float32),
            jax.ShapeDtypeStruct((1, 1), jnp.float32),
            jax.ShapeDtypeStruct((1, NI), jnp.float32),
        ],
        scratch_shapes=[pltpu.SMEM((1,), jnp.float32)],
    )(scores, rating_matrix, pid2)

    full = lambda shape: pl.BlockSpec(shape, lambda i: (0,) * len(shape))
    loss, uf = pl.pallas_call(
        _mlp_loss_body,
        grid=(nblk,),
        in_specs=[
            pl.BlockSpec((UBLK, NI), lambda i: (i, 0)),
            full((NI, 256)), full((1, 256)),
            full((256, 128)), full((1, 128)),
            full((128, NI)), full((1, NI)),
            full((NI, 128)), full((1, 128)),
            full((128, NI)), full((1, NI)),
            full((NI, 128)), full((1, 128)),
            full((128, NI)), full((1, NI)),
            full((1, NI)),
            pl.BlockSpec((UBLK, 1), lambda i: (i, 0)),
            full((1, 1)),
        ],
        out_specs=[
            pl.BlockSpec((1, 1), lambda i: (0, 0)),
            pl.BlockSpec((UBLK, 1), lambda i: (i, 0)),
        ],
        out_shape=[
            jax.ShapeDtypeStruct((1, 1), jnp.float32),
            jax.ShapeDtypeStruct((NU, 1), jnp.float32),
        ],
    )(rating_matrix, eW1, eb1.reshape(1, -1), eW2, eb2.reshape(1, -1),
      eW3, eb3.reshape(1, -1), pW1, pb1.reshape(1, -1), pW2,
      pb2.reshape(1, -1), uW1, ub1.reshape(1, -1), uW2, ub2.reshape(1, -1),
      ipw, recip, rmean)

    return (loss.reshape(()), ipw.reshape(NI), uf.reshape(NU),
            expo.reshape(NI))
